# baseline (device time: 106146 ns/iter reference)
import jax
import jax.numpy as jnp
from jax import lax
from jax.experimental import pallas as pl
from jax.experimental.pallas import tpu as pltpu

N_DEV = 4
TOK = 512
D = 512
F = 1024
E_LOC = 2


def kernel(x, assign, W1, W2):
    assign2d = assign.reshape(TOK, 1)

    def body(x_ref, a_ref, w1_ref, w2_ref, out_ref,
             xall, aall, contrib, rbuf,
             xs_sems, xr_sems, as_sems, ar_sems, rss_sems, rsr_sems):
        my = lax.axis_index("i")
        left = (my + N_DEV - 1) % N_DEV
        right = (my + 1) % N_DEV

        barrier_sem = pltpu.get_barrier_semaphore()
        for nbr in (left, right):
            pl.semaphore_signal(
                barrier_sem, inc=1,
                device_id=(nbr,), device_id_type=pl.DeviceIdType.MESH,
            )
        pl.semaphore_wait(barrier_sem, 2)

        xall[pl.ds(my * TOK, TOK), :] = x_ref[...]
        aall[pl.ds(my * TOK, TOK), :] = a_ref[...]

        for h in range(N_DEV - 1):
            o_send = (my + N_DEV - h) % N_DEV
            rdma_x = pltpu.make_async_remote_copy(
                src_ref=xall.at[pl.ds(o_send * TOK, TOK), :],
                dst_ref=xall.at[pl.ds(o_send * TOK, TOK), :],
                send_sem=xs_sems.at[h],
                recv_sem=xr_sems.at[h],
                device_id=(right,),
                device_id_type=pl.DeviceIdType.MESH,
            )
            rdma_a = pltpu.make_async_remote_copy(
                src_ref=aall.at[pl.ds(o_send * TOK, TOK), :],
                dst_ref=aall.at[pl.ds(o_send * TOK, TOK), :],
                send_sem=as_sems.at[h],
                recv_sem=ar_sems.at[h],
                device_id=(right,),
                device_id_type=pl.DeviceIdType.MESH,
            )
            rdma_x.start()
            rdma_a.start()
            rdma_x.wait()
            rdma_a.wait()

        a_full = aall[...]
        for blk in range(N_DEV):
            xb = xall[pl.ds(blk * TOK, TOK), :]
            ab = a_full[blk * TOK:(blk + 1) * TOK, :]
            acc = jnp.zeros((TOK, D), dtype=jnp.float32)
            for e in range(E_LOC):
                h_act = jnp.maximum(
                    jnp.dot(xb, w1_ref[e], preferred_element_type=jnp.float32),
                    0.0,
                )
                y = jnp.dot(h_act, w2_ref[e], preferred_element_type=jnp.float32)
                e_glob = my * E_LOC + e
                acc = acc + jnp.where(ab == e_glob, y, 0.0)
            contrib[pl.ds(blk * TOK, TOK), :] = acc

        for s in range(N_DEV - 1):
            b_send = (my + N_DEV - 1 - s) % N_DEV
            b_recv = (my + N_DEV - 2 - s) % N_DEV
            rdma = pltpu.make_async_remote_copy(
                src_ref=contrib.at[pl.ds(b_send * TOK, TOK), :],
                dst_ref=rbuf.at[s],
                send_sem=rss_sems.at[s],
                recv_sem=rsr_sems.at[s],
                device_id=(right,),
                device_id_type=pl.DeviceIdType.MESH,
            )
            rdma.start()
            rdma.wait()
            if s < N_DEV - 2:
                contrib[pl.ds(b_recv * TOK, TOK), :] = (
                    contrib[pl.ds(b_recv * TOK, TOK), :] + rbuf[s]
                )
            else:
                out_ref[...] = contrib[pl.ds(my * TOK, TOK), :] + rbuf[s]

    return pl.pallas_call(
        body,
        out_shape=jax.ShapeDtypeStruct((TOK, D), jnp.float32),
        in_specs=[
            pl.BlockSpec(memory_space=pltpu.VMEM),
            pl.BlockSpec(memory_space=pltpu.VMEM),
            pl.BlockSpec(memory_space=pltpu.VMEM),
            pl.BlockSpec(memory_space=pltpu.VMEM),
        ],
        out_specs=pl.BlockSpec(memory_space=pltpu.VMEM),
        scratch_shapes=[
            pltpu.VMEM((N_DEV * TOK, D), jnp.float32),
            pltpu.VMEM((N_DEV * TOK, 1), jnp.int32),
            pltpu.VMEM((N_DEV * TOK, D), jnp.float32),
            pltpu.VMEM((N_DEV - 1, TOK, D), jnp.float32),
            pltpu.SemaphoreType.DMA((N_DEV - 1,)),
            pltpu.SemaphoreType.DMA((N_DEV - 1,)),
            pltpu.SemaphoreType.DMA((N_DEV - 1,)),
            pltpu.SemaphoreType.DMA((N_DEV - 1,)),
            pltpu.SemaphoreType.DMA((N_DEV - 1,)),
            pltpu.SemaphoreType.DMA((N_DEV - 1,)),
        ],
        compiler_params=pltpu.CompilerParams(collective_id=0),
    )(x, assign2d, W1, W2)


# device time: 91879 ns/iter; 1.1553x vs baseline; 1.1553x over previous
import jax
import jax.numpy as jnp
from jax import lax
from jax.experimental import pallas as pl
from jax.experimental.pallas import tpu as pltpu

N_DEV = 4
TOK = 512
D = 512
F = 1024
E_LOC = 2


def kernel(x, assign, W1, W2):
    assign2d = assign.reshape(TOK, 1)

    def body(x_ref, a_ref, w1_ref, w2_ref, out_ref,
             xall, aall, contrib, rbuf,
             xs_sems, xr_sems, as_sems, ar_sems, rss_sems, rsr_sems):
        my = lax.axis_index("i")
        left = (my + N_DEV - 1) % N_DEV
        right = (my + 1) % N_DEV

        barrier_sem = pltpu.get_barrier_semaphore()
        for nbr in (left, right):
            pl.semaphore_signal(
                barrier_sem, inc=1,
                device_id=(nbr,), device_id_type=pl.DeviceIdType.MESH,
            )
        pl.semaphore_wait(barrier_sem, 2)

        def ag_hop(h):
            o_send = (my + N_DEV - h) % N_DEV
            rdma_x = pltpu.make_async_remote_copy(
                src_ref=xall.at[pl.ds(o_send * TOK, TOK), :],
                dst_ref=xall.at[pl.ds(o_send * TOK, TOK), :],
                send_sem=xs_sems.at[h],
                recv_sem=xr_sems.at[h],
                device_id=(right,),
                device_id_type=pl.DeviceIdType.MESH,
            )
            rdma_a = pltpu.make_async_remote_copy(
                src_ref=aall.at[pl.ds(o_send * TOK, TOK), :],
                dst_ref=aall.at[pl.ds(o_send * TOK, TOK), :],
                send_sem=as_sems.at[h],
                recv_sem=ar_sems.at[h],
                device_id=(right,),
                device_id_type=pl.DeviceIdType.MESH,
            )
            rdma_x.start()
            rdma_a.start()
            return rdma_x, rdma_a

        def rs_step(s):
            b_send = (my + N_DEV - 1 - s) % N_DEV
            rdma = pltpu.make_async_remote_copy(
                src_ref=contrib.at[pl.ds(b_send * TOK, TOK), :],
                dst_ref=rbuf.at[s],
                send_sem=rss_sems.at[s],
                recv_sem=rsr_sems.at[s],
                device_id=(right,),
                device_id_type=pl.DeviceIdType.MESH,
            )
            rdma.start()
            return rdma

        def compute_block(b):
            xb = xall[pl.ds(b * TOK, TOK), :]
            ab = aall[pl.ds(b * TOK, TOK), :]
            acc = jnp.zeros((TOK, D), dtype=jnp.float32)
            for e in range(E_LOC):
                h_act = jnp.maximum(
                    jnp.dot(xb, w1_ref[e], preferred_element_type=jnp.float32),
                    0.0,
                )
                y = jnp.dot(h_act, w2_ref[e], preferred_element_type=jnp.float32)
                acc = acc + jnp.where(ab == my * E_LOC + e, y, 0.0)
            contrib[pl.ds(b * TOK, TOK), :] = acc

        xall[pl.ds(my * TOK, TOK), :] = x_ref[...]
        aall[pl.ds(my * TOK, TOK), :] = a_ref[...]
        ag0_x, ag0_a = ag_hop(0)
        compute_block(my)

        ag0_x.wait()
        ag0_a.wait()
        b0 = (my + N_DEV - 1) % N_DEV
        ag1_x, ag1_a = ag_hop(1)
        compute_block(b0)
        rs0 = rs_step(0)

        ag1_x.wait()
        ag1_a.wait()
        b1 = (my + N_DEV - 2) % N_DEV
        ag2_x, ag2_a = ag_hop(2)
        compute_block(b1)
        rs0.wait()
        contrib[pl.ds(b1 * TOK, TOK), :] = (
            contrib[pl.ds(b1 * TOK, TOK), :] + rbuf[0]
        )
        rs1 = rs_step(1)

        ag2_x.wait()
        ag2_a.wait()
        b2 = (my + N_DEV - 3) % N_DEV
        compute_block(b2)
        rs1.wait()
        contrib[pl.ds(b2 * TOK, TOK), :] = (
            contrib[pl.ds(b2 * TOK, TOK), :] + rbuf[1]
        )
        rs2 = rs_step(2)

        rs2.wait()
        out_ref[...] = contrib[pl.ds(my * TOK, TOK), :] + rbuf[2]

    return pl.pallas_call(
        body,
        out_shape=jax.ShapeDtypeStruct((TOK, D), jnp.float32),
        in_specs=[
            pl.BlockSpec(memory_space=pltpu.VMEM),
            pl.BlockSpec(memory_space=pltpu.VMEM),
            pl.BlockSpec(memory_space=pltpu.VMEM),
            pl.BlockSpec(memory_space=pltpu.VMEM),
        ],
        out_specs=pl.BlockSpec(memory_space=pltpu.VMEM),
        scratch_shapes=[
            pltpu.VMEM((N_DEV * TOK, D), jnp.float32),
            pltpu.VMEM((N_DEV * TOK, 1), jnp.int32),
            pltpu.VMEM((N_DEV * TOK, D), jnp.float32),
            pltpu.VMEM((N_DEV - 1, TOK, D), jnp.float32),
            pltpu.SemaphoreType.DMA((N_DEV - 1,)),
            pltpu.SemaphoreType.DMA((N_DEV - 1,)),
            pltpu.SemaphoreType.DMA((N_DEV - 1,)),
            pltpu.SemaphoreType.DMA((N_DEV - 1,)),
            pltpu.SemaphoreType.DMA((N_DEV - 1,)),
            pltpu.SemaphoreType.DMA((N_DEV - 1,)),
        ],
        compiler_params=pltpu.CompilerParams(collective_id=0),
    )(x, assign2d, W1, W2)


# device time: 60608 ns/iter; 1.7514x vs baseline; 1.5160x over previous
import jax
import jax.numpy as jnp
from jax import lax
from jax.experimental import pallas as pl
from jax.experimental.pallas import tpu as pltpu

N_DEV = 4
TOK = 512
HALF = TOK // 2
D = 512
F = 1024
E_LOC = 2


def kernel(x, assign, W1, W2):
    assign2d = assign.reshape(TOK, 1)

    def body(x_ref, a_ref, w1_ref, w2_ref, out_ref,
             xall, aall, contrib, rtop, rbot, rfin_l, rfin_r,
             ag_s, ag_r, as_s, as_r, rs_s, rs_r):
        my = lax.axis_index("i")
        left = (my + N_DEV - 1) % N_DEV
        right = (my + 1) % N_DEV
        opp = (my + 2) % N_DEV

        barrier_sem = pltpu.get_barrier_semaphore()
        for nbr in (left, right):
            pl.semaphore_signal(
                barrier_sem, inc=1,
                device_id=(nbr,), device_id_type=pl.DeviceIdType.MESH,
            )
        pl.semaphore_wait(barrier_sem, 2)

        def copy(src_ref, dst_ref, send_sem, recv_sem, dst_dev):
            rdma = pltpu.make_async_remote_copy(
                src_ref=src_ref, dst_ref=dst_ref,
                send_sem=send_sem, recv_sem=recv_sem,
                device_id=(dst_dev,), device_id_type=pl.DeviceIdType.MESH,
            )
            rdma.start()
            return rdma

        def xa_pair(src_row, n_rows, dst_row, k, dst_dev):
            cx = copy(xall.at[pl.ds(src_row, n_rows), :],
                      xall.at[pl.ds(dst_row, n_rows), :],
                      ag_s.at[k], ag_r.at[k], dst_dev)
            ca = copy(aall.at[pl.ds(src_row, n_rows), :],
                      aall.at[pl.ds(dst_row, n_rows), :],
                      as_s.at[k], as_r.at[k], dst_dev)
            return cx, ca

        def compute_block(b):
            xb = xall[pl.ds(b * TOK, TOK), :]
            ab = aall[pl.ds(b * TOK, TOK), :]
            acc = jnp.zeros((TOK, D), dtype=jnp.float32)
            for e in range(E_LOC):
                h_act = jnp.maximum(
                    jnp.dot(xb, w1_ref[e], preferred_element_type=jnp.float32),
                    0.0,
                )
                y = jnp.dot(h_act, w2_ref[e], preferred_element_type=jnp.float32)
                acc = acc + jnp.where(ab == my * E_LOC + e, y, 0.0)
            contrib[pl.ds(b * TOK, TOK), :] = acc

        xall[pl.ds(my * TOK, TOK), :] = x_ref[...]
        aall[pl.ds(my * TOK, TOK), :] = a_ref[...]
        g0 = xa_pair(my * TOK, TOK, my * TOK, 0, right)
        g1 = xa_pair(my * TOK, TOK, my * TOK, 1, left)
        compute_block(my)

        for c in g0:
            c.wait()
        g2 = xa_pair(left * TOK, HALF, left * TOK, 2, right)
        compute_block(left)
        for c in g1:
            c.wait()
        g3 = xa_pair(right * TOK + HALF, HALF, right * TOK + HALF, 3, left)
        compute_block(right)

        for c in g2 + g3:
            c.wait()
        compute_block(opp)

        r0 = copy(contrib.at[pl.ds(opp * TOK, HALF), :], rtop,
                  rs_s.at[0], rs_r.at[0], right)
        r1 = copy(contrib.at[pl.ds(opp * TOK + HALF, HALF), :], rbot,
                  rs_s.at[1], rs_r.at[1], left)

        r0.wait()
        contrib[pl.ds(right * TOK, HALF), :] = (
            contrib[pl.ds(right * TOK, HALF), :] + rtop[...]
        )
        r2 = copy(contrib.at[pl.ds(right * TOK, TOK), :], rfin_l,
                  rs_s.at[2], rs_r.at[2], right)

        r1.wait()
        contrib[pl.ds(left * TOK + HALF, HALF), :] = (
            contrib[pl.ds(left * TOK + HALF, HALF), :] + rbot[...]
        )
        r3 = copy(contrib.at[pl.ds(left * TOK, TOK), :], rfin_r,
                  rs_s.at[3], rs_r.at[3], left)

        r2.wait()
        r3.wait()
        out_ref[...] = (
            contrib[pl.ds(my * TOK, TOK), :] + rfin_l[...] + rfin_r[...]
        )

    return pl.pallas_call(
        body,
        out_shape=jax.ShapeDtypeStruct((TOK, D), jnp.float32),
        in_specs=[
            pl.BlockSpec(memory_space=pltpu.VMEM),
            pl.BlockSpec(memory_space=pltpu.VMEM),
            pl.BlockSpec(memory_space=pltpu.VMEM),
            pl.BlockSpec(memory_space=pltpu.VMEM),
        ],
        out_specs=pl.BlockSpec(memory_space=pltpu.VMEM),
        scratch_shapes=[
            pltpu.VMEM((N_DEV * TOK, D), jnp.float32),
            pltpu.VMEM((N_DEV * TOK, 1), jnp.int32),
            pltpu.VMEM((N_DEV * TOK, D), jnp.float32),
            pltpu.VMEM((HALF, D), jnp.float32),
            pltpu.VMEM((HALF, D), jnp.float32),
            pltpu.VMEM((TOK, D), jnp.float32),
            pltpu.VMEM((TOK, D), jnp.float32),
            pltpu.SemaphoreType.DMA((4,)),
            pltpu.SemaphoreType.DMA((4,)),
            pltpu.SemaphoreType.DMA((4,)),
            pltpu.SemaphoreType.DMA((4,)),
            pltpu.SemaphoreType.DMA((4,)),
            pltpu.SemaphoreType.DMA((4,)),
        ],
        compiler_params=pltpu.CompilerParams(collective_id=0),
    )(x, assign2d, W1, W2)


# device time: 42482 ns/iter; 2.4986x vs baseline; 1.4267x over previous
import jax
import jax.numpy as jnp
from jax import lax
from jax.experimental import pallas as pl
from jax.experimental.pallas import tpu as pltpu

N_DEV = 4
TOK = 512
HALF = TOK // 2
D = 512
F = 1024
E_LOC = 2


def kernel(x, assign, W1, W2):
    assign2d = assign.reshape(TOK, 1)
    x = x.astype(jnp.bfloat16)
    W1 = W1.astype(jnp.bfloat16)
    W2 = W2.astype(jnp.bfloat16)

    def body(x_ref, a_ref, w1_ref, w2_ref, out_ref,
             xall, aall, contrib, rtop, rbot, rfin_l, rfin_r,
             ag_s, ag_r, as_s, as_r, rs_s, rs_r):
        my = lax.axis_index("i")
        left = (my + N_DEV - 1) % N_DEV
        right = (my + 1) % N_DEV
        opp = (my + 2) % N_DEV

        barrier_sem = pltpu.get_barrier_semaphore()
        for nbr in (left, right):
            pl.semaphore_signal(
                barrier_sem, inc=1,
                device_id=(nbr,), device_id_type=pl.DeviceIdType.MESH,
            )
        pl.semaphore_wait(barrier_sem, 2)

        def copy(src_ref, dst_ref, send_sem, recv_sem, dst_dev):
            rdma = pltpu.make_async_remote_copy(
                src_ref=src_ref, dst_ref=dst_ref,
                send_sem=send_sem, recv_sem=recv_sem,
                device_id=(dst_dev,), device_id_type=pl.DeviceIdType.MESH,
            )
            rdma.start()
            return rdma

        def xa_pair(src_row, n_rows, dst_row, k, dst_dev):
            cx = copy(xall.at[pl.ds(src_row, n_rows), :],
                      xall.at[pl.ds(dst_row, n_rows), :],
                      ag_s.at[k], ag_r.at[k], dst_dev)
            ca = copy(aall.at[pl.ds(src_row, n_rows), :],
                      aall.at[pl.ds(dst_row, n_rows), :],
                      as_s.at[k], as_r.at[k], dst_dev)
            return cx, ca

        def compute_block(b):
            xb = xall[pl.ds(b * TOK, TOK), :]
            ab = aall[pl.ds(b * TOK, TOK), :]
            acc = jnp.zeros((TOK, D), dtype=jnp.float32)
            for e in range(E_LOC):
                h_act = jnp.maximum(
                    jnp.dot(xb, w1_ref[e], preferred_element_type=jnp.float32),
                    0.0,
                ).astype(jnp.bfloat16)
                y = jnp.dot(h_act, w2_ref[e], preferred_element_type=jnp.float32)
                acc = acc + jnp.where(ab == my * E_LOC + e, y, 0.0)
            contrib[pl.ds(b * TOK, TOK), :] = acc.astype(jnp.bfloat16)

        xall[pl.ds(my * TOK, TOK), :] = x_ref[...]
        aall[pl.ds(my * TOK, TOK), :] = a_ref[...]
        g0 = xa_pair(my * TOK, TOK, my * TOK, 0, right)
        g1 = xa_pair(my * TOK, TOK, my * TOK, 1, left)
        compute_block(my)

        for c in g0:
            c.wait()
        g2 = xa_pair(left * TOK, HALF, left * TOK, 2, right)
        compute_block(left)
        for c in g1:
            c.wait()
        g3 = xa_pair(right * TOK + HALF, HALF, right * TOK + HALF, 3, left)
        compute_block(right)

        for c in g2 + g3:
            c.wait()
        compute_block(opp)

        r0 = copy(contrib.at[pl.ds(opp * TOK, HALF), :], rtop,
                  rs_s.at[0], rs_r.at[0], right)
        r1 = copy(contrib.at[pl.ds(opp * TOK + HALF, HALF), :], rbot,
                  rs_s.at[1], rs_r.at[1], left)

        r0.wait()
        contrib[pl.ds(right * TOK, HALF), :] = (
            contrib[pl.ds(right * TOK, HALF), :] + rtop[...]
        )
        r2 = copy(contrib.at[pl.ds(right * TOK, TOK), :], rfin_l,
                  rs_s.at[2], rs_r.at[2], right)

        r1.wait()
        contrib[pl.ds(left * TOK + HALF, HALF), :] = (
            contrib[pl.ds(left * TOK + HALF, HALF), :] + rbot[...]
        )
        r3 = copy(contrib.at[pl.ds(left * TOK, TOK), :], rfin_r,
                  rs_s.at[3], rs_r.at[3], left)

        r2.wait()
        r3.wait()
        out_ref[...] = (
            contrib[pl.ds(my * TOK, TOK), :].astype(jnp.float32)
            + rfin_l[...].astype(jnp.float32)
            + rfin_r[...].astype(jnp.float32)
        )

    return pl.pallas_call(
        body,
        out_shape=jax.ShapeDtypeStruct((TOK, D), jnp.float32),
        in_specs=[
            pl.BlockSpec(memory_space=pltpu.VMEM),
            pl.BlockSpec(memory_space=pltpu.VMEM),
            pl.BlockSpec(memory_space=pltpu.VMEM),
            pl.BlockSpec(memory_space=pltpu.VMEM),
        ],
        out_specs=pl.BlockSpec(memory_space=pltpu.VMEM),
        scratch_shapes=[
            pltpu.VMEM((N_DEV * TOK, D), jnp.bfloat16),
            pltpu.VMEM((N_DEV * TOK, 1), jnp.int32),
            pltpu.VMEM((N_DEV * TOK, D), jnp.bfloat16),
            pltpu.VMEM((HALF, D), jnp.bfloat16),
            pltpu.VMEM((HALF, D), jnp.bfloat16),
            pltpu.VMEM((TOK, D), jnp.bfloat16),
            pltpu.VMEM((TOK, D), jnp.bfloat16),
            pltpu.SemaphoreType.DMA((4,)),
            pltpu.SemaphoreType.DMA((4,)),
            pltpu.SemaphoreType.DMA((4,)),
            pltpu.SemaphoreType.DMA((4,)),
            pltpu.SemaphoreType.DMA((4,)),
            pltpu.SemaphoreType.DMA((4,)),
        ],
        compiler_params=pltpu.CompilerParams(collective_id=0),
    )(x, assign2d, W1, W2)
